# baseline (speedup 1.0000x reference)
import jax
import jax.numpy as jnp
from jax.experimental import pallas as pl
from jax.experimental.pallas import tpu as pltpu

_B, _N, _D, _H = 8, 65536, 3, 64
_BLK = 8192
_NB = _N // _BLK


def _topo_kernel(s_ref, t_ref, w1a_ref, w2_ref, b2_ref, out_ref, dacc):
    b = pl.program_id(0)
    nb = pl.program_id(1)

    @pl.when(jnp.logical_and(b == 0, nb == 0))
    def _init():
        dacc[...] = jnp.zeros_like(dacc)

    w1a = w1a_ref[...]
    w2 = w2_ref[...]
    b2 = b2_ref[...]

    def agg(x):
        ones = jnp.ones((1, _BLK), dtype=jnp.float32)
        xa = jnp.concatenate([x, ones], axis=0)
        h = jax.lax.dot_general(w1a, xa, (((1,), (0,)), ((), ())),
                                preferred_element_type=jnp.float32)
        h = jnp.maximum(h, 0.0)
        o = jax.lax.dot_general(w2, h, (((1,), (0,)), ((), ())),
                                preferred_element_type=jnp.float32)
        o = jnp.maximum(o + b2, 0.0)
        return jnp.sum(o, keepdims=True)

    part = agg(s_ref[0]) - agg(t_ref[0])
    dacc[pl.ds(b, 1), :] += part

    @pl.when(jnp.logical_and(b == _B - 1, nb == _NB - 1))
    def _fin():
        d = dacc[...]
        out_ref[...] = jnp.mean(d * d, keepdims=True)


def kernel(student_diagrams, teacher_diagrams, W1, b1, W2, b2):
    sT = jnp.swapaxes(student_diagrams, 1, 2)
    tT = jnp.swapaxes(teacher_diagrams, 1, 2)
    w1a = jnp.concatenate([W1.T, b1[:, None]], axis=1)
    w2r = W2.T
    b2r = b2.reshape(1, 1)

    out = pl.pallas_call(
        _topo_kernel,
        grid=(_B, _NB),
        in_specs=[
            pl.BlockSpec((1, _D, _BLK), lambda b, nb: (b, 0, nb)),
            pl.BlockSpec((1, _D, _BLK), lambda b, nb: (b, 0, nb)),
            pl.BlockSpec((_H, _D + 1), lambda b, nb: (0, 0)),
            pl.BlockSpec((1, _H), lambda b, nb: (0, 0)),
            pl.BlockSpec((1, 1), lambda b, nb: (0, 0)),
        ],
        out_specs=pl.BlockSpec((1, 1), lambda b, nb: (0, 0)),
        out_shape=jax.ShapeDtypeStruct((1, 1), jnp.float32),
        scratch_shapes=[
            pltpu.VMEM((_B, 1), jnp.float32),
        ],
    )(sT, tT, w1a, w2r, b2r)
    return out[0, 0]
